# baseline (speedup 1.0000x reference)
import jax
import jax.numpy as jnp
from jax import lax
from jax.experimental import pallas as pl
from jax.experimental.pallas import tpu as pltpu
from jax.experimental.pallas import tpu_sc as plsc

_H, _W = 384, 512
_B, _C, _S = 2, 2, 2
_P = _H * _W
_NUM_IDS = 9
_I = _NUM_IDS - 1
_SCALE = 64.0
_RESCALE = 1.0 / 64.0
_EMB = (64.0, 64.0)

_NW = 32
_WPB = _NW // _B
_CH = _P // _WPB
_NQ = 5


def _sc_body(lab_hbm, sig_hbm, off_hbm, out_hbm,
             lab0_v, lab1_v, sig_v, off0_v, off1_v, part_v, *accs):
    cid = lax.axis_index("c")
    sid = lax.axis_index("s")
    wid = sid * 2 + cid
    b = wid // _WPB
    base_pix = (wid % _WPB) * _CH

    pltpu.sync_copy(lab_hbm.at[b, 0, pl.ds(base_pix, _CH)], lab0_v)
    pltpu.sync_copy(lab_hbm.at[b, 1, pl.ds(base_pix, _CH)], lab1_v)
    pltpu.sync_copy(sig_hbm.at[b, pl.ds(base_pix, _CH)], sig_v)
    pltpu.sync_copy(off_hbm.at[b, 0, pl.ds(base_pix, _CH)], off0_v)
    pltpu.sync_copy(off_hbm.at[b, 1, pl.ds(base_pix, _CH)], off1_v)

    for a in accs:
        for l in range(16):
            a[pl.ds(l * 16, 16)] = jnp.zeros((16,), jnp.float32)

    lane = lax.iota(jnp.int32, 16)
    lane16 = lane * 16
    ones = jnp.full((16,), 1.0, jnp.float32)

    def step(s, _):
        o = s * 16
        pvec = base_pix + o + lane
        sg = sig_v[pl.ds(o, 16)]
        sgn = _SCALE / (1.0 + jnp.exp(sg * (-1.0 / (2.0 * _SCALE))))
        sgc = sgn - 32.0
        row = jnp.right_shift(pvec, 9).astype(jnp.float32)
        col = jnp.bitwise_and(pvec, 511).astype(jnp.float32)
        e0 = (row * (_EMB[0] / _H) + off0_v[pl.ds(o, 16)]) * _RESCALE
        e1 = (col * (_EMB[1] / _W) + off1_v[pl.ds(o, 16)]) * _RESCALE
        vals = (ones, sgc, sgc * sgc, e0, e1)
        for ci, lab_v in ((0, lab0_v), (1, lab1_v)):
            idx = lane16 + lab_v[pl.ds(o, 16)]
            for qi in range(_NQ):
                plsc.addupdate_scatter(accs[ci * _NQ + qi], [idx], vals[qi])
        return ()

    lax.fori_loop(0, _CH // 16, step, ())

    for ci in range(_C):
        for qi in range(_NQ):
            a = accs[ci * _NQ + qi]
            bins = a[pl.ds(0, 16)]
            for l in range(1, 16):
                bins = bins + a[pl.ds(l * 16, 16)]
            part_v[pl.ds((ci * _NQ + qi) * 16, 16)] = bins

    pltpu.sync_copy(part_v, out_hbm.at[b, wid % _WPB])


def _phase_a_sc(labels, sigma_map, offset_map):
    lab_f = labels.reshape(_B, _C, _P)
    sig_f = sigma_map.reshape(_B, _P)
    off_f = offset_map.reshape(_B, _S, _P)
    mesh = plsc.VectorSubcoreMesh(core_axis_name="c", subcore_axis_name="s")
    scratch = [
        pltpu.VMEM((_CH,), jnp.int32),
        pltpu.VMEM((_CH,), jnp.int32),
        pltpu.VMEM((_CH,), jnp.float32),
        pltpu.VMEM((_CH,), jnp.float32),
        pltpu.VMEM((_CH,), jnp.float32),
        pltpu.VMEM((_C * _NQ * 16,), jnp.float32),
    ] + [pltpu.VMEM((256,), jnp.float32) for _ in range(_C * _NQ)]
    fn = pl.kernel(
        _sc_body,
        mesh=mesh,
        out_type=jax.ShapeDtypeStruct((_B, _WPB, _C * _NQ * 16), jnp.float32),
        scratch_types=scratch,
        compiler_params=pltpu.CompilerParams(needs_layout_passes=False),
    )
    return fn(lab_f, sig_f, off_f).reshape(_B, _WPB, _C, _NQ, 16)


def _tc_body(seed_ref, off_ref, lab_ref, sig_ref, part_ref, dists_ref, sums_ref):
    b = pl.program_id(0)
    c = pl.program_id(1)
    lab = lab_ref[0, 0]
    sig = _SCALE / (1.0 + jnp.exp(sig_ref[0, 0] * (-1.0 / (2.0 * _SCALE))))
    row = lax.broadcasted_iota(jnp.int32, (_H, _W), 0).astype(jnp.float32)
    col = lax.broadcasted_iota(jnp.int32, (_H, _W), 1).astype(jnp.float32)
    e0 = (row * (_EMB[0] / _H) + off_ref[0, 0]) * _RESCALE
    e1 = (col * (_EMB[1] / _W) + off_ref[0, 1]) * _RESCALE
    seed = seed_ref[0, 0]

    sums = jnp.sum(part_ref[0, :, 0], axis=0)

    inst_sum = jnp.zeros((), jnp.float32)
    smooth_sum = jnp.zeros((), jnp.float32)
    seed_mask = jnp.zeros((_H, _W), jnp.float32)
    for i in range(_I):
        cnt = sums[0, i + 1]
        smc = sums[1, i + 1] / cnt
        ssq = sums[2, i + 1]
        m0 = sums[3, i + 1] / cnt
        m1 = sums[4, i + 1] / cnt
        sm = 32.0 + smc
        inv2s2 = 1.0 / (2.0 * sm * sm)
        m = lab == (i + 1)
        d0 = e0 - m0
        d1 = e1 - m1
        d = (d0 * d0 + d1 * d1) * inv2s2
        phi = jnp.exp(-d)

        @pl.when(c == _C - 1)
        def _store():
            dists_ref[0, i] = phi

        logp = jnp.maximum(-d, -100.0)
        log1mp = jnp.maximum(jnp.log(1.0 - phi), -100.0)
        inst_sum = inst_sum - (jnp.sum(log1mp)
                               + jnp.sum(jnp.where(m, logp - log1mp, 0.0)))
        smooth_sum = smooth_sum + (ssq / cnt - smc * smc)
        seed_mask = seed_mask + jnp.where(m, phi, 0.0)

    dseed = seed - seed_mask
    seed_sum = jnp.sum(dseed * dseed) * (1.0 / (_H * _W))

    cur = jnp.stack([
        jnp.full((8, 128), inst_sum, jnp.float32),
        jnp.full((8, 128), smooth_sum, jnp.float32),
        jnp.full((8, 128), seed_sum, jnp.float32),
    ])
    first = jnp.logical_and(b == 0, c == 0)

    @pl.when(first)
    def _init():
        sums_ref[...] = cur

    @pl.when(jnp.logical_not(first))
    def _acc():
        sums_ref[...] = sums_ref[...] + cur


def _phase_b_tc(seed_map, offset_map, labels, sigma_map, partials):
    dists, sums = pl.pallas_call(
        _tc_body,
        grid=(_B, _C),
        in_specs=[
            pl.BlockSpec((1, 1, _H, _W), lambda b, c: (b, c, 0, 0)),
            pl.BlockSpec((1, _S, _H, _W), lambda b, c: (b, 0, 0, 0)),
            pl.BlockSpec((1, 1, _H, _W), lambda b, c: (b, c, 0, 0)),
            pl.BlockSpec((1, 1, _H, _W), lambda b, c: (b, 0, 0, 0)),
            pl.BlockSpec((1, _WPB, 1, _NQ, 16), lambda b, c: (b, 0, c, 0, 0)),
        ],
        out_specs=[
            pl.BlockSpec((1, _I, _H, _W), lambda b, c: (b, 0, 0, 0)),
            pl.BlockSpec((3, 8, 128), lambda b, c: (0, 0, 0)),
        ],
        out_shape=[
            jax.ShapeDtypeStruct((_B, _I, _H, _W), jnp.float32),
            jax.ShapeDtypeStruct((3, 8, 128), jnp.float32),
        ],
    )(seed_map, offset_map, labels, sigma_map, partials)
    return dists, sums


def kernel(seed_map, offset_map, labels, sigma_map):
    labels = labels.astype(jnp.int32)
    partials = _phase_a_sc(labels, sigma_map, offset_map)
    dists, sums = _phase_b_tc(seed_map, offset_map, labels, sigma_map, partials)
    s = sums[:, 0, 0]
    n = float(_B * _C * _I)
    il = s[0] / n
    sl = s[1] / n
    sel = s[2]
    loss = (il + sl + sel) * (1.0 / (_B * _C))
    stats = jnp.stack([il, sl, sel])
    return loss, dists, stats


# speedup vs baseline: 1.1591x; 1.1591x over previous
import jax
import jax.numpy as jnp
from jax import lax
from jax.experimental import pallas as pl
from jax.experimental.pallas import tpu as pltpu
from jax.experimental.pallas import tpu_sc as plsc

_H, _W = 384, 512
_B, _C, _S = 2, 2, 2
_P = _H * _W
_NUM_IDS = 9
_I = _NUM_IDS - 1
_SCALE = 64.0
_RESCALE = 1.0 / 64.0
_EMB = (64.0, 64.0)

_NW = 32
_WPB = _NW // _B
_CH = _P // _WPB
_NQ = 5


def _sc_body(lab_hbm, sig_hbm, off_hbm, out_hbm,
             lab0_v, lab1_v, sig_v, off0_v, off1_v, part_v, *accs):
    cid = lax.axis_index("c")
    sid = lax.axis_index("s")
    wid = sid * 2 + cid
    b = wid // _WPB
    base_pix = (wid % _WPB) * _CH

    pltpu.sync_copy(lab_hbm.at[b, 0, pl.ds(base_pix, _CH)], lab0_v)
    pltpu.sync_copy(lab_hbm.at[b, 1, pl.ds(base_pix, _CH)], lab1_v)
    pltpu.sync_copy(sig_hbm.at[b, pl.ds(base_pix, _CH)], sig_v)
    pltpu.sync_copy(off_hbm.at[b, 0, pl.ds(base_pix, _CH)], off0_v)
    pltpu.sync_copy(off_hbm.at[b, 1, pl.ds(base_pix, _CH)], off1_v)

    for a in accs:
        for l in range(16):
            a[pl.ds(l * 16, 16)] = jnp.zeros((16,), jnp.float32)

    lane = lax.iota(jnp.int32, 16)
    lane16 = lane * 16
    ones = jnp.full((16,), 1.0, jnp.float32)

    @plsc.parallel_loop(0, _CH // 16, unroll=8)
    def _step(s):
        o = s * 16
        pvec = base_pix + o + lane
        sg = sig_v[pl.ds(o, 16)]
        sgn = _SCALE / (1.0 + jnp.exp(sg * (-1.0 / (2.0 * _SCALE))))
        sgc = sgn - 32.0
        row = jnp.right_shift(pvec, 9).astype(jnp.float32)
        col = jnp.bitwise_and(pvec, 511).astype(jnp.float32)
        e0 = (row * (_EMB[0] / _H) + off0_v[pl.ds(o, 16)]) * _RESCALE
        e1 = (col * (_EMB[1] / _W) + off1_v[pl.ds(o, 16)]) * _RESCALE
        vals = (ones, sgc, sgc * sgc, e0, e1)
        for ci, lab_v in ((0, lab0_v), (1, lab1_v)):
            idx = lane16 + lab_v[pl.ds(o, 16)]
            for qi in range(_NQ):
                plsc.addupdate_scatter(accs[ci * _NQ + qi], [idx], vals[qi])

    for ci in range(_C):
        for qi in range(_NQ):
            a = accs[ci * _NQ + qi]
            bins = a[pl.ds(0, 16)]
            for l in range(1, 16):
                bins = bins + a[pl.ds(l * 16, 16)]
            part_v[pl.ds((ci * _NQ + qi) * 16, 16)] = bins

    pltpu.sync_copy(part_v, out_hbm.at[b, wid % _WPB])


def _phase_a_sc(labels, sigma_map, offset_map):
    lab_f = labels.reshape(_B, _C, _P)
    sig_f = sigma_map.reshape(_B, _P)
    off_f = offset_map.reshape(_B, _S, _P)
    mesh = plsc.VectorSubcoreMesh(core_axis_name="c", subcore_axis_name="s")
    scratch = [
        pltpu.VMEM((_CH,), jnp.int32),
        pltpu.VMEM((_CH,), jnp.int32),
        pltpu.VMEM((_CH,), jnp.float32),
        pltpu.VMEM((_CH,), jnp.float32),
        pltpu.VMEM((_CH,), jnp.float32),
        pltpu.VMEM((_C * _NQ * 16,), jnp.float32),
    ] + [pltpu.VMEM((256,), jnp.float32) for _ in range(_C * _NQ)]
    fn = pl.kernel(
        _sc_body,
        mesh=mesh,
        out_type=jax.ShapeDtypeStruct((_B, _WPB, _C * _NQ * 16), jnp.float32),
        scratch_types=scratch,
        compiler_params=pltpu.CompilerParams(needs_layout_passes=False),
    )
    return fn(lab_f, sig_f, off_f).reshape(_B, _WPB, _C, _NQ, 16)


def _tc_body(seed_ref, off_ref, lab_ref, sig_ref, part_ref, dists_ref, sums_ref):
    b = pl.program_id(0)
    c = pl.program_id(1)
    lab = lab_ref[0, 0]
    sig = _SCALE / (1.0 + jnp.exp(sig_ref[0, 0] * (-1.0 / (2.0 * _SCALE))))
    row = lax.broadcasted_iota(jnp.int32, (_H, _W), 0).astype(jnp.float32)
    col = lax.broadcasted_iota(jnp.int32, (_H, _W), 1).astype(jnp.float32)
    e0 = (row * (_EMB[0] / _H) + off_ref[0, 0]) * _RESCALE
    e1 = (col * (_EMB[1] / _W) + off_ref[0, 1]) * _RESCALE
    seed = seed_ref[0, 0]

    sums = jnp.sum(part_ref[0, :, 0], axis=0)

    inst_sum = jnp.zeros((), jnp.float32)
    smooth_sum = jnp.zeros((), jnp.float32)
    seed_mask = jnp.zeros((_H, _W), jnp.float32)
    for i in range(_I):
        cnt = sums[0, i + 1]
        smc = sums[1, i + 1] / cnt
        ssq = sums[2, i + 1]
        m0 = sums[3, i + 1] / cnt
        m1 = sums[4, i + 1] / cnt
        sm = 32.0 + smc
        inv2s2 = 1.0 / (2.0 * sm * sm)
        m = lab == (i + 1)
        d0 = e0 - m0
        d1 = e1 - m1
        d = (d0 * d0 + d1 * d1) * inv2s2
        phi = jnp.exp(-d)

        @pl.when(c == _C - 1)
        def _store():
            dists_ref[0, i] = phi

        logp = jnp.maximum(-d, -100.0)
        log1mp = jnp.maximum(jnp.log(1.0 - phi), -100.0)
        inst_sum = inst_sum - (jnp.sum(log1mp)
                               + jnp.sum(jnp.where(m, logp - log1mp, 0.0)))
        smooth_sum = smooth_sum + (ssq / cnt - smc * smc)
        seed_mask = seed_mask + jnp.where(m, phi, 0.0)

    dseed = seed - seed_mask
    seed_sum = jnp.sum(dseed * dseed) * (1.0 / (_H * _W))

    cur = jnp.stack([
        jnp.full((8, 128), inst_sum, jnp.float32),
        jnp.full((8, 128), smooth_sum, jnp.float32),
        jnp.full((8, 128), seed_sum, jnp.float32),
    ])
    first = jnp.logical_and(b == 0, c == 0)

    @pl.when(first)
    def _init():
        sums_ref[...] = cur

    @pl.when(jnp.logical_not(first))
    def _acc():
        sums_ref[...] = sums_ref[...] + cur


def _phase_b_tc(seed_map, offset_map, labels, sigma_map, partials):
    dists, sums = pl.pallas_call(
        _tc_body,
        grid=(_B, _C),
        in_specs=[
            pl.BlockSpec((1, 1, _H, _W), lambda b, c: (b, c, 0, 0)),
            pl.BlockSpec((1, _S, _H, _W), lambda b, c: (b, 0, 0, 0)),
            pl.BlockSpec((1, 1, _H, _W), lambda b, c: (b, c, 0, 0)),
            pl.BlockSpec((1, 1, _H, _W), lambda b, c: (b, 0, 0, 0)),
            pl.BlockSpec((1, _WPB, 1, _NQ, 16), lambda b, c: (b, 0, c, 0, 0)),
        ],
        out_specs=[
            pl.BlockSpec((1, _I, _H, _W), lambda b, c: (b, 0, 0, 0)),
            pl.BlockSpec((3, 8, 128), lambda b, c: (0, 0, 0)),
        ],
        out_shape=[
            jax.ShapeDtypeStruct((_B, _I, _H, _W), jnp.float32),
            jax.ShapeDtypeStruct((3, 8, 128), jnp.float32),
        ],
    )(seed_map, offset_map, labels, sigma_map, partials)
    return dists, sums


def kernel(seed_map, offset_map, labels, sigma_map):
    labels = labels.astype(jnp.int32)
    partials = _phase_a_sc(labels, sigma_map, offset_map)
    dists, sums = _phase_b_tc(seed_map, offset_map, labels, sigma_map, partials)
    s = sums[:, 0, 0]
    n = float(_B * _C * _I)
    il = s[0] / n
    sl = s[1] / n
    sel = s[2]
    loss = (il + sl + sel) * (1.0 / (_B * _C))
    stats = jnp.stack([il, sl, sel])
    return loss, dists, stats


# speedup vs baseline: 1.2276x; 1.0591x over previous
import jax
import jax.numpy as jnp
from jax import lax
from jax.experimental import pallas as pl
from jax.experimental.pallas import tpu as pltpu
from jax.experimental.pallas import tpu_sc as plsc

_H, _W = 384, 512
_B, _C, _S = 2, 2, 2
_P = _H * _W
_NUM_IDS = 9
_I = _NUM_IDS - 1
_SCALE = 64.0
_RESCALE = 1.0 / 64.0
_EMB = (64.0, 64.0)

_NW = 32
_WPB = _NW // _B
_CH = _P // _WPB
_NQ = 5


def _sc_body(lab_hbm, sig_hbm, off_hbm, out_hbm,
             lab0_v, lab1_v, sig_v, off0_v, off1_v, *accs):
    cid = lax.axis_index("c")
    sid = lax.axis_index("s")
    wid = sid * 2 + cid
    b = wid // _WPB
    base_pix = (wid % _WPB) * _CH

    pltpu.sync_copy(lab_hbm.at[b, 0, pl.ds(base_pix, _CH)], lab0_v)
    pltpu.sync_copy(lab_hbm.at[b, 1, pl.ds(base_pix, _CH)], lab1_v)
    pltpu.sync_copy(sig_hbm.at[b, pl.ds(base_pix, _CH)], sig_v)
    pltpu.sync_copy(off_hbm.at[b, 0, pl.ds(base_pix, _CH)], off0_v)
    pltpu.sync_copy(off_hbm.at[b, 1, pl.ds(base_pix, _CH)], off1_v)

    for a in accs:
        for l in range(16):
            a[pl.ds(l * 16, 16)] = jnp.zeros((16,), jnp.float32)

    lane = lax.iota(jnp.int32, 16)
    ones = jnp.full((16,), 1.0, jnp.float32)

    @plsc.parallel_loop(0, _CH // 16, unroll=8)
    def _step(s):
        o = s * 16
        pvec = base_pix + o + lane
        sg = sig_v[pl.ds(o, 16)]
        sgn = _SCALE / (1.0 + jnp.exp(sg * (-1.0 / (2.0 * _SCALE))))
        sgc = sgn - 32.0
        row = jnp.right_shift(pvec, 9).astype(jnp.float32)
        col = jnp.bitwise_and(pvec, 511).astype(jnp.float32)
        e0 = (row * (_EMB[0] / _H) + off0_v[pl.ds(o, 16)]) * _RESCALE
        e1 = (col * (_EMB[1] / _W) + off1_v[pl.ds(o, 16)]) * _RESCALE
        vals = (ones, sgc, sgc * sgc, e0, e1)
        for ci, lab_v in ((0, lab0_v), (1, lab1_v)):
            idx = jnp.left_shift(lab_v[pl.ds(o, 16)], 4) + lane
            for qi in range(_NQ):
                plsc.addupdate_scatter(accs[ci * _NQ + qi], [idx], vals[qi])

    for ai in range(_C * _NQ):
        pltpu.sync_copy(accs[ai], out_hbm.at[b, wid % _WPB, pl.ds(ai * 256, 256)])


def _phase_a_sc(labels, sigma_map, offset_map):
    lab_f = labels.reshape(_B, _C, _P)
    sig_f = sigma_map.reshape(_B, _P)
    off_f = offset_map.reshape(_B, _S, _P)
    mesh = plsc.VectorSubcoreMesh(core_axis_name="c", subcore_axis_name="s")
    scratch = [
        pltpu.VMEM((_CH,), jnp.int32),
        pltpu.VMEM((_CH,), jnp.int32),
        pltpu.VMEM((_CH,), jnp.float32),
        pltpu.VMEM((_CH,), jnp.float32),
        pltpu.VMEM((_CH,), jnp.float32),
    ] + [pltpu.VMEM((256,), jnp.float32) for _ in range(_C * _NQ)]
    fn = pl.kernel(
        _sc_body,
        mesh=mesh,
        out_type=jax.ShapeDtypeStruct((_B, _WPB, _C * _NQ * 256), jnp.float32),
        scratch_types=scratch,
        compiler_params=pltpu.CompilerParams(needs_layout_passes=False),
    )
    return fn(lab_f, sig_f, off_f).reshape(_B, _WPB, _C, _NQ, 16, 16)


def _tc_body(seed_ref, off_ref, lab_ref, sig_ref, part_ref, dists_ref, sums_ref):
    b = pl.program_id(0)
    c = pl.program_id(1)
    lab = lab_ref[0, 0]
    sig = _SCALE / (1.0 + jnp.exp(sig_ref[0, 0] * (-1.0 / (2.0 * _SCALE))))
    row = lax.broadcasted_iota(jnp.int32, (_H, _W), 0).astype(jnp.float32)
    col = lax.broadcasted_iota(jnp.int32, (_H, _W), 1).astype(jnp.float32)
    e0 = (row * (_EMB[0] / _H) + off_ref[0, 0]) * _RESCALE
    e1 = (col * (_EMB[1] / _W) + off_ref[0, 1]) * _RESCALE
    seed = seed_ref[0, 0]

    sums = jnp.sum(part_ref[0, :, 0], axis=(0, 3))

    inst_sum = jnp.zeros((), jnp.float32)
    smooth_sum = jnp.zeros((), jnp.float32)
    seed_mask = jnp.zeros((_H, _W), jnp.float32)
    for i in range(_I):
        cnt = sums[0, i + 1]
        smc = sums[1, i + 1] / cnt
        ssq = sums[2, i + 1]
        m0 = sums[3, i + 1] / cnt
        m1 = sums[4, i + 1] / cnt
        sm = 32.0 + smc
        inv2s2 = 1.0 / (2.0 * sm * sm)
        m = lab == (i + 1)
        d0 = e0 - m0
        d1 = e1 - m1
        d = (d0 * d0 + d1 * d1) * inv2s2
        phi = jnp.exp(-d)

        @pl.when(c == _C - 1)
        def _store():
            dists_ref[0, i] = phi

        logp = jnp.maximum(-d, -100.0)
        log1mp = jnp.maximum(jnp.log(1.0 - phi), -100.0)
        inst_sum = inst_sum - (jnp.sum(log1mp)
                               + jnp.sum(jnp.where(m, logp - log1mp, 0.0)))
        smooth_sum = smooth_sum + (ssq / cnt - smc * smc)
        seed_mask = seed_mask + jnp.where(m, phi, 0.0)

    dseed = seed - seed_mask
    seed_sum = jnp.sum(dseed * dseed) * (1.0 / (_H * _W))

    cur = jnp.stack([
        jnp.full((8, 128), inst_sum, jnp.float32),
        jnp.full((8, 128), smooth_sum, jnp.float32),
        jnp.full((8, 128), seed_sum, jnp.float32),
    ])
    first = jnp.logical_and(b == 0, c == 0)

    @pl.when(first)
    def _init():
        sums_ref[...] = cur

    @pl.when(jnp.logical_not(first))
    def _acc():
        sums_ref[...] = sums_ref[...] + cur


def _phase_b_tc(seed_map, offset_map, labels, sigma_map, partials):
    dists, sums = pl.pallas_call(
        _tc_body,
        grid=(_B, _C),
        in_specs=[
            pl.BlockSpec((1, 1, _H, _W), lambda b, c: (b, c, 0, 0)),
            pl.BlockSpec((1, _S, _H, _W), lambda b, c: (b, 0, 0, 0)),
            pl.BlockSpec((1, 1, _H, _W), lambda b, c: (b, c, 0, 0)),
            pl.BlockSpec((1, 1, _H, _W), lambda b, c: (b, 0, 0, 0)),
            pl.BlockSpec((1, _WPB, 1, _NQ, 16, 16),
                         lambda b, c: (b, 0, c, 0, 0, 0)),
        ],
        out_specs=[
            pl.BlockSpec((1, _I, _H, _W), lambda b, c: (b, 0, 0, 0)),
            pl.BlockSpec((3, 8, 128), lambda b, c: (0, 0, 0)),
        ],
        out_shape=[
            jax.ShapeDtypeStruct((_B, _I, _H, _W), jnp.float32),
            jax.ShapeDtypeStruct((3, 8, 128), jnp.float32),
        ],
    )(seed_map, offset_map, labels, sigma_map, partials)
    return dists, sums


def kernel(seed_map, offset_map, labels, sigma_map):
    labels = labels.astype(jnp.int32)
    partials = _phase_a_sc(labels, sigma_map, offset_map)
    dists, sums = _phase_b_tc(seed_map, offset_map, labels, sigma_map, partials)
    s = sums[:, 0, 0]
    n = float(_B * _C * _I)
    il = s[0] / n
    sl = s[1] / n
    sel = s[2]
    loss = (il + sl + sel) * (1.0 / (_B * _C))
    stats = jnp.stack([il, sl, sel])
    return loss, dists, stats


# speedup vs baseline: 1.2348x; 1.0059x over previous
import jax
import jax.numpy as jnp
from jax import lax
from jax.experimental import pallas as pl
from jax.experimental.pallas import tpu as pltpu
from jax.experimental.pallas import tpu_sc as plsc

_H, _W = 384, 512
_B, _C, _S = 2, 2, 2
_P = _H * _W
_NUM_IDS = 9
_I = _NUM_IDS - 1
_SCALE = 64.0
_RESCALE = 1.0 / 64.0
_EMB = (64.0, 64.0)

_NW = 32
_WPB = _NW // _B
_CH = _P // _WPB
_NQ = 5


def _sc_body(lab_hbm, sig_hbm, off_hbm, out_hbm,
             lab0_v, lab1_v, sig_v, off0_v, off1_v, sem_a, sem_b, *accs):
    cid = lax.axis_index("c")
    sid = lax.axis_index("s")
    wid = sid * 2 + cid
    b = wid // _WPB
    base_pix = (wid % _WPB) * _CH

    half = _CH // 2
    srcs = (lab_hbm.at[b, 0], lab_hbm.at[b, 1], sig_hbm.at[b],
            off_hbm.at[b, 0], off_hbm.at[b, 1])
    dsts = (lab0_v, lab1_v, sig_v, off0_v, off1_v)
    cps = []
    for h, sem in ((0, sem_a), (1, sem_b)):
        for src, dst in zip(srcs, dsts):
            cps.append(pltpu.async_copy(
                src.at[pl.ds(base_pix + h * half, half)],
                dst.at[pl.ds(h * half, half)], sem))

    for a in accs:
        for l in range(16):
            a[pl.ds(l * 16, 16)] = jnp.zeros((16,), jnp.float32)

    lane = lax.iota(jnp.int32, 16)
    ones = jnp.full((16,), 1.0, jnp.float32)

    def make_loop(lo, hi):
        @plsc.parallel_loop(lo, hi, unroll=4)
        def _step(s):
            for g in range(2):
                o = s * 32 + g * 16
                pvec = base_pix + o + lane
                sg = sig_v[pl.ds(o, 16)]
                sgn = _SCALE / (1.0 + jnp.exp(sg * (-1.0 / (2.0 * _SCALE))))
                sgc = sgn - 32.0
                row = jnp.right_shift(pvec, 9).astype(jnp.float32)
                col = jnp.bitwise_and(pvec, 511).astype(jnp.float32)
                e0 = (row * (_EMB[0] / _H) + off0_v[pl.ds(o, 16)]) * _RESCALE
                e1 = (col * (_EMB[1] / _W) + off1_v[pl.ds(o, 16)]) * _RESCALE
                vals = (ones, sgc, sgc * sgc, e0, e1)
                for ci, lab_v in ((0, lab0_v), (1, lab1_v)):
                    idx = jnp.left_shift(lab_v[pl.ds(o, 16)], 4) + lane
                    for qi in range(_NQ):
                        plsc.addupdate_scatter(
                            accs[g * _C * _NQ + ci * _NQ + qi], [idx],
                            vals[qi])

    for c_ in cps[:5]:
        c_.wait()
    make_loop(0, half // 32)
    for c_ in cps[5:]:
        c_.wait()
    make_loop(half // 32, _CH // 32)

    for ai in range(2 * _C * _NQ):
        pltpu.sync_copy(accs[ai], out_hbm.at[b, wid % _WPB, pl.ds(ai * 256, 256)])


def _phase_a_sc(labels, sigma_map, offset_map):
    lab_f = labels.reshape(_B, _C, _P)
    sig_f = sigma_map.reshape(_B, _P)
    off_f = offset_map.reshape(_B, _S, _P)
    mesh = plsc.VectorSubcoreMesh(core_axis_name="c", subcore_axis_name="s")
    scratch = [
        pltpu.VMEM((_CH,), jnp.int32),
        pltpu.VMEM((_CH,), jnp.int32),
        pltpu.VMEM((_CH,), jnp.float32),
        pltpu.VMEM((_CH,), jnp.float32),
        pltpu.VMEM((_CH,), jnp.float32),
        pltpu.SemaphoreType.DMA,
        pltpu.SemaphoreType.DMA,
    ] + [pltpu.VMEM((256,), jnp.float32) for _ in range(2 * _C * _NQ)]
    fn = pl.kernel(
        _sc_body,
        mesh=mesh,
        out_type=jax.ShapeDtypeStruct((_B, _WPB, 2 * _C * _NQ * 256),
                                      jnp.float32),
        scratch_types=scratch,
        compiler_params=pltpu.CompilerParams(needs_layout_passes=False),
    )
    return fn(lab_f, sig_f, off_f).reshape(_B, 2 * _WPB, _C, _NQ, 16, 16)


def _tc_body(seed_ref, off_ref, lab_ref, sig_ref, part_ref, dists_ref, sums_ref):
    b = pl.program_id(0)
    c = pl.program_id(1)
    lab = lab_ref[0, 0]
    sig = _SCALE / (1.0 + jnp.exp(sig_ref[0, 0] * (-1.0 / (2.0 * _SCALE))))
    row = lax.broadcasted_iota(jnp.int32, (_H, _W), 0).astype(jnp.float32)
    col = lax.broadcasted_iota(jnp.int32, (_H, _W), 1).astype(jnp.float32)
    e0 = (row * (_EMB[0] / _H) + off_ref[0, 0]) * _RESCALE
    e1 = (col * (_EMB[1] / _W) + off_ref[0, 1]) * _RESCALE
    seed = seed_ref[0, 0]

    sums = jnp.sum(part_ref[0, :, 0], axis=(0, 3))

    inst_sum = jnp.zeros((), jnp.float32)
    smooth_sum = jnp.zeros((), jnp.float32)
    seed_mask = jnp.zeros((_H, _W), jnp.float32)
    for i in range(_I):
        cnt = sums[0, i + 1]
        smc = sums[1, i + 1] / cnt
        ssq = sums[2, i + 1]
        m0 = sums[3, i + 1] / cnt
        m1 = sums[4, i + 1] / cnt
        sm = 32.0 + smc
        inv2s2 = 1.0 / (2.0 * sm * sm)
        m = lab == (i + 1)
        d0 = e0 - m0
        d1 = e1 - m1
        d = (d0 * d0 + d1 * d1) * inv2s2
        phi = jnp.exp(-d)

        @pl.when(c == _C - 1)
        def _store():
            dists_ref[0, i] = phi

        logp = jnp.maximum(-d, -100.0)
        log1mp = jnp.maximum(jnp.log(1.0 - phi), -100.0)
        inst_sum = inst_sum - (jnp.sum(log1mp)
                               + jnp.sum(jnp.where(m, logp - log1mp, 0.0)))
        smooth_sum = smooth_sum + (ssq / cnt - smc * smc)
        seed_mask = seed_mask + jnp.where(m, phi, 0.0)

    dseed = seed - seed_mask
    seed_sum = jnp.sum(dseed * dseed) * (1.0 / (_H * _W))

    cur = jnp.stack([
        jnp.full((8, 128), inst_sum, jnp.float32),
        jnp.full((8, 128), smooth_sum, jnp.float32),
        jnp.full((8, 128), seed_sum, jnp.float32),
    ])
    first = jnp.logical_and(b == 0, c == 0)

    @pl.when(first)
    def _init():
        sums_ref[...] = cur

    @pl.when(jnp.logical_not(first))
    def _acc():
        sums_ref[...] = sums_ref[...] + cur


def _phase_b_tc(seed_map, offset_map, labels, sigma_map, partials):
    dists, sums = pl.pallas_call(
        _tc_body,
        grid=(_B, _C),
        in_specs=[
            pl.BlockSpec((1, 1, _H, _W), lambda b, c: (b, c, 0, 0)),
            pl.BlockSpec((1, _S, _H, _W), lambda b, c: (b, 0, 0, 0)),
            pl.BlockSpec((1, 1, _H, _W), lambda b, c: (b, c, 0, 0)),
            pl.BlockSpec((1, 1, _H, _W), lambda b, c: (b, 0, 0, 0)),
            pl.BlockSpec((1, 2 * _WPB, 1, _NQ, 16, 16),
                         lambda b, c: (b, 0, c, 0, 0, 0)),
        ],
        out_specs=[
            pl.BlockSpec((1, _I, _H, _W), lambda b, c: (b, 0, 0, 0)),
            pl.BlockSpec((3, 8, 128), lambda b, c: (0, 0, 0)),
        ],
        out_shape=[
            jax.ShapeDtypeStruct((_B, _I, _H, _W), jnp.float32),
            jax.ShapeDtypeStruct((3, 8, 128), jnp.float32),
        ],
    )(seed_map, offset_map, labels, sigma_map, partials)
    return dists, sums


def kernel(seed_map, offset_map, labels, sigma_map):
    labels = labels.astype(jnp.int32)
    partials = _phase_a_sc(labels, sigma_map, offset_map)
    dists, sums = _phase_b_tc(seed_map, offset_map, labels, sigma_map, partials)
    s = sums[:, 0, 0]
    n = float(_B * _C * _I)
    il = s[0] / n
    sl = s[1] / n
    sel = s[2]
    loss = (il + sl + sel) * (1.0 / (_B * _C))
    stats = jnp.stack([il, sl, sel])
    return loss, dists, stats


# speedup vs baseline: 1.4950x; 1.2108x over previous
import jax
import jax.numpy as jnp
from jax import lax
from jax.experimental import pallas as pl
from jax.experimental.pallas import tpu as pltpu
from jax.experimental.pallas import tpu_sc as plsc

_H, _W = 384, 512
_B, _C, _S = 2, 2, 2
_P = _H * _W
_NUM_IDS = 9
_I = _NUM_IDS - 1
_SCALE = 64.0
_RESCALE = 1.0 / 64.0
_EMB = (64.0, 64.0)

_NW = 32
_WPB = _NW // _B
_CH = _P // _WPB
_NQ = 5


_RPW = _H // _WPB


def _sc_body(lab_hbm, sig_hbm, off_hbm, out_hbm,
             lab0_v, lab1_v, sig_v, off0_v, off1_v, sem_a, sem_b, *accs):
    cid = lax.axis_index("c")
    sid = lax.axis_index("s")
    wid = sid * 2 + cid
    b = wid // _WPB
    wq = wid % _WPB
    h0 = wq * _RPW

    srcs = (lab_hbm.at[b, 0], lab_hbm.at[b, 1], sig_hbm.at[b, 0],
            off_hbm.at[b, 0], off_hbm.at[b, 1])
    dsts = (lab0_v, lab1_v, sig_v, off0_v, off1_v)
    cps = []
    for lo, n, sem in ((0, 8, sem_a), (8, 16, sem_b)):
        for src, dst in zip(srcs, dsts):
            cps.append(pltpu.async_copy(
                src.at[pl.ds(h0 + lo, n), :],
                dst.at[pl.ds(lo, n), :], sem))

    for a in accs:
        for l in range(16):
            a[l, :] = jnp.zeros((16,), jnp.float32)

    lane = lax.iota(jnp.int32, 16)
    ones = jnp.full((16,), 1.0, jnp.float32)

    def make_loop(lo, hi):
        @plsc.parallel_loop(lo, hi, unroll=4)
        def _step(s):
            r = jnp.right_shift(s, 4)
            rowf = (h0 + r).astype(jnp.float32)
            for g in range(2):
                cb = jnp.bitwise_and(s, 15) * 32 + g * 16
                sg = sig_v[r, pl.ds(cb, 16)]
                sgn = _SCALE / (1.0 + jnp.exp(sg * (-1.0 / (2.0 * _SCALE))))
                sgc = sgn - 32.0
                colf = (cb + lane).astype(jnp.float32)
                e0 = (rowf * (_EMB[0] / _H) + off0_v[r, pl.ds(cb, 16)]) * _RESCALE
                e1 = (colf * (_EMB[1] / _W) + off1_v[r, pl.ds(cb, 16)]) * _RESCALE
                vals = (ones, sgc, sgc * sgc, e0, e1)
                for ci, lab_v in ((0, lab0_v), (1, lab1_v)):
                    labv = lab_v[r, pl.ds(cb, 16)]
                    for qi in range(_NQ):
                        plsc.addupdate_scatter(
                            accs[ci * 2 * _NQ + g * _NQ + qi], [labv, lane],
                            vals[qi])

    ngrp = _RPW * (_W // 32)
    for c_ in cps[:5]:
        c_.wait()
    make_loop(0, 8 * (_W // 32))
    for c_ in cps[5:]:
        c_.wait()
    make_loop(8 * (_W // 32), ngrp)

    for ci in range(_C):
        for k in range(2 * _NQ):
            pltpu.sync_copy(accs[ci * 2 * _NQ + k], out_hbm.at[b, wq, ci, k])


def _phase_a_sc(labels, sigma_map, offset_map):
    mesh = plsc.VectorSubcoreMesh(core_axis_name="c", subcore_axis_name="s")
    scratch = [
        pltpu.VMEM((_RPW, _W), jnp.int32),
        pltpu.VMEM((_RPW, _W), jnp.int32),
        pltpu.VMEM((_RPW, _W), jnp.float32),
        pltpu.VMEM((_RPW, _W), jnp.float32),
        pltpu.VMEM((_RPW, _W), jnp.float32),
        pltpu.SemaphoreType.DMA,
        pltpu.SemaphoreType.DMA,
    ] + [pltpu.VMEM((16, 16), jnp.float32) for _ in range(2 * _C * _NQ)]
    fn = pl.kernel(
        _sc_body,
        mesh=mesh,
        out_type=jax.ShapeDtypeStruct((_B, _WPB, _C, 2 * _NQ, 16, 16),
                                      jnp.float32),
        scratch_types=scratch,
        compiler_params=pltpu.CompilerParams(needs_layout_passes=False),
    )
    return fn(labels, sigma_map, offset_map)


def _tc_body(seed_ref, off_ref, lab_ref, sig_ref, part_ref, dists_ref, sums_ref):
    b = pl.program_id(0)
    c = pl.program_id(1)
    lab = lab_ref[0, 0]
    sig = _SCALE / (1.0 + jnp.exp(sig_ref[0, 0] * (-1.0 / (2.0 * _SCALE))))
    row = lax.broadcasted_iota(jnp.int32, (_H, _W), 0).astype(jnp.float32)
    col = lax.broadcasted_iota(jnp.int32, (_H, _W), 1).astype(jnp.float32)
    e0 = (row * (_EMB[0] / _H) + off_ref[0, 0]) * _RESCALE
    e1 = (col * (_EMB[1] / _W) + off_ref[0, 1]) * _RESCALE
    seed = seed_ref[0, 0]

    w = jnp.sum(part_ref[0, :, 0], axis=(0, 3))
    sums = w[0:_NQ] + w[_NQ:2 * _NQ]

    inst_sum = jnp.zeros((), jnp.float32)
    smooth_sum = jnp.zeros((), jnp.float32)
    seed_mask = jnp.zeros((_H, _W), jnp.float32)
    for i in range(_I):
        cnt = sums[0, i + 1]
        smc = sums[1, i + 1] / cnt
        ssq = sums[2, i + 1]
        m0 = sums[3, i + 1] / cnt
        m1 = sums[4, i + 1] / cnt
        sm = 32.0 + smc
        inv2s2 = 1.0 / (2.0 * sm * sm)
        m = lab == (i + 1)
        d0 = e0 - m0
        d1 = e1 - m1
        d = (d0 * d0 + d1 * d1) * inv2s2
        phi = jnp.exp(-d)

        @pl.when(c == _C - 1)
        def _store():
            dists_ref[0, i] = phi

        logp = jnp.maximum(-d, -100.0)
        log1mp = jnp.maximum(jnp.log(1.0 - phi), -100.0)
        inst_sum = inst_sum - (jnp.sum(log1mp)
                               + jnp.sum(jnp.where(m, logp - log1mp, 0.0)))
        smooth_sum = smooth_sum + (ssq / cnt - smc * smc)
        seed_mask = seed_mask + jnp.where(m, phi, 0.0)

    dseed = seed - seed_mask
    seed_sum = jnp.sum(dseed * dseed) * (1.0 / (_H * _W))

    cur = jnp.stack([
        jnp.full((8, 128), inst_sum, jnp.float32),
        jnp.full((8, 128), smooth_sum, jnp.float32),
        jnp.full((8, 128), seed_sum, jnp.float32),
    ])
    first = jnp.logical_and(b == 0, c == 0)

    @pl.when(first)
    def _init():
        sums_ref[...] = cur

    @pl.when(jnp.logical_not(first))
    def _acc():
        sums_ref[...] = sums_ref[...] + cur


def _phase_b_tc(seed_map, offset_map, labels, sigma_map, partials):
    dists, sums = pl.pallas_call(
        _tc_body,
        grid=(_B, _C),
        in_specs=[
            pl.BlockSpec((1, 1, _H, _W), lambda b, c: (b, c, 0, 0)),
            pl.BlockSpec((1, _S, _H, _W), lambda b, c: (b, 0, 0, 0)),
            pl.BlockSpec((1, 1, _H, _W), lambda b, c: (b, c, 0, 0)),
            pl.BlockSpec((1, 1, _H, _W), lambda b, c: (b, 0, 0, 0)),
            pl.BlockSpec((1, _WPB, 1, 2 * _NQ, 16, 16),
                         lambda b, c: (b, 0, c, 0, 0, 0)),
        ],
        out_specs=[
            pl.BlockSpec((1, _I, _H, _W), lambda b, c: (b, 0, 0, 0)),
            pl.BlockSpec((3, 8, 128), lambda b, c: (0, 0, 0)),
        ],
        out_shape=[
            jax.ShapeDtypeStruct((_B, _I, _H, _W), jnp.float32),
            jax.ShapeDtypeStruct((3, 8, 128), jnp.float32),
        ],
    )(seed_map, offset_map, labels, sigma_map, partials)
    return dists, sums


def kernel(seed_map, offset_map, labels, sigma_map):
    labels = labels.astype(jnp.int32)
    partials = _phase_a_sc(labels, sigma_map, offset_map)
    dists, sums = _phase_b_tc(seed_map, offset_map, labels, sigma_map, partials)
    s = sums[:, 0, 0]
    n = float(_B * _C * _I)
    il = s[0] / n
    sl = s[1] / n
    sel = s[2]
    loss = (il + sl + sel) * (1.0 / (_B * _C))
    stats = jnp.stack([il, sl, sel])
    return loss, dists, stats
